# Initial kernel scaffold; baseline (speedup 1.0000x reference)
import jax
import jax.numpy as jnp
from jax.experimental import pallas as pl


def kernel(p, r, edge_index, pq_rel, params):
    raise NotImplementedError("write your pallas kernel here")


# speedup vs baseline: 7.4551x; 7.4551x over previous
import functools

import jax
import jax.numpy as jnp
from jax import lax
from jax.experimental import pallas as pl
from jax.experimental.pallas import tpu as pltpu
from jax.experimental.pallas import tpu_sc as plsc


def _sc_gather_rows(table, idx_pad):
    n_pad = idx_pad.shape[0]
    d = table.shape[1]
    info = plsc.get_sparse_core_info()
    nc, ns = info.num_cores, info.num_subcores
    nw = nc * ns
    b_per_w = n_pad // nw
    mesh = plsc.VectorSubcoreMesh(core_axis_name="c", subcore_axis_name="s")

    @functools.partial(
        pl.kernel,
        mesh=mesh,
        out_type=jax.ShapeDtypeStruct((n_pad, d), jnp.float32),
        scratch_types=[
            pltpu.VMEM((b_per_w,), jnp.int32),
            pltpu.VMEM((b_per_w, d), jnp.float32),
            pltpu.SemaphoreType.DMA,
        ],
    )
    def gather_k(table_hbm, idx_hbm, out_hbm, idx_v, rows_v, sem):
        wid = lax.axis_index("s") * nc + lax.axis_index("c")
        base = wid * b_per_w
        pltpu.sync_copy(idx_hbm.at[pl.ds(base, b_per_w)], idx_v)
        pltpu.async_copy(table_hbm.at[idx_v], rows_v, sem).wait()
        pltpu.sync_copy(rows_v, out_hbm.at[pl.ds(base, b_per_w)])

    return gather_k(table, idx_pad)


def _step_body(Q, B, E, H,
               qcol_ref, rcol_ref, ei_t_ref, ei_ref,
               qemb_ref, u0_ref, du_ref, init_ref,
               w1s_ref, b1s_ref, w2s_ref, b2s_ref,
               ws_ref, wd_ref, bias1_ref, w2blk_ref, bias2_ref,
               wi_ref, bi_ref, wh_ref, bh_ref,
               w1p_ref, b1p_ref, vw_ref, biasq_ref,
               y_ref,
               ht_ref, gsrc_ref, gdst_ref, gdt_ref, gst_ref):
    t = pl.program_id(0)
    N = Q * B

    @pl.when(t == 0)
    def _init():
        ht_ref[...] = init_ref[...]
        src_col = ei_t_ref[:, 0:1]
        dst_col = ei_t_ref[:, 1:2]
        iota_eq = lax.broadcasted_iota(jnp.int32, (E, Q), 1)
        gsrc_ref[...] = (iota_eq == src_col).astype(jnp.float32)
        gdst_ref[...] = (iota_eq == dst_col).astype(jnp.float32)
        src_row = ei_ref[0:1, :]
        dst_row = ei_ref[1:2, :]
        iota_qe = lax.broadcasted_iota(jnp.int32, (Q, E), 0)
        gdt_ref[...] = (iota_qe == dst_row).astype(jnp.float32)
        gst_ref[...] = (iota_qe == src_row).astype(jnp.float32)

    ht = ht_ref[...]
    qcol = qcol_ref[0]
    rcol = rcol_ref[0]

    feat = qemb_ref[...] + qcol * (u0_ref[...] + rcol * du_ref[...])
    m2 = jnp.concatenate([ht, feat], axis=1)

    dot = lambda a, b: jnp.dot(a, b, preferred_element_type=jnp.float32)

    h1s = jnp.maximum(dot(m2, w1s_ref[...]) + b1s_ref[...], 0.0)
    m_self = dot(h1s, w2s_ref[...]) + b2s_ref[...]

    ps = dot(m2, ws_ref[...])
    pd = dot(m2, wd_ref[...])
    ps_v = ps.reshape(Q, B * 2 * H)
    pd_v = pd.reshape(Q, B * 2 * H)
    pre_v = dot(gsrc_ref[...], ps_v) + dot(gdst_ref[...], pd_v)
    h1m = jnp.maximum(pre_v.reshape(E * B, 2 * H) + bias1_ref[...], 0.0)
    msgs = dot(h1m, w2blk_ref[...]) + bias2_ref[...]

    msgs_v = msgs.reshape(E, B * 2 * H)
    agg_o = dot(gdt_ref[...], msgs_v)
    agg_i = dot(gst_ref[...], msgs_v)
    lane = lax.broadcasted_iota(jnp.int32, (1, B * 2 * H), 1)
    out_half = (lane % (2 * H)) < H
    agg_full = jnp.where(out_half, agg_o, agg_i).reshape(N, 2 * H)
    agg = agg_full[:, :H] + agg_full[:, H:]

    ht_ = m_self + agg

    hcat = jnp.concatenate([ht, ht_], axis=1)
    gi = dot(hcat, wi_ref[...]) + bi_ref[...]
    gh = dot(ht, wh_ref[...]) + bh_ref[...]
    rr = jax.nn.sigmoid(gi[:, :H] + gh[:, :H])
    zz = jax.nn.sigmoid(gi[:, H:2 * H] + gh[:, H:2 * H])
    nn_ = jnp.tanh(gi[:, 2 * H:] + rr * gh[:, 2 * H:])
    hcand = (1.0 - zz) * nn_ + zz * ht
    hnew = qcol * hcand + (1.0 - qcol) * ht
    ht_ref[...] = hnew

    pin = jnp.concatenate([hnew, qemb_ref[...]], axis=1)
    h1p = jnp.maximum(dot(pin, w1p_ref[...]) + b1p_ref[...], 0.0)
    logit = dot(h1p, vw_ref[...]) + biasq_ref[...]
    y_ref[...] = jax.nn.sigmoid(logit).reshape(1, N, 1)


def _run_scan(L, Q, B, E, H, ops, interpret=False):
    N = Q * B
    full = lambda shape: pl.BlockSpec(shape, lambda t: (0,) * len(shape))
    per_t = lambda shape: pl.BlockSpec((1,) + shape[1:], lambda t: (t,) + (0,) * (len(shape) - 1))
    in_specs = [
        per_t((L, N, 1)),
        per_t((L, N, 1)),
        full((E, 2)),
        full((2, E)),
        full((N, H)),
        full((N, H)),
        full((N, H)),
        full((N, H)),
        full((2 * H, H)), full((1, H)), full((H, H)), full((1, H)),
        full((2 * H, 2 * H)), full((2 * H, 2 * H)), full((1, 2 * H)),
        full((2 * H, 2 * H)), full((1, 2 * H)),
        full((2 * H, 3 * H)), full((1, 3 * H)),
        full((H, 3 * H)), full((1, 3 * H)),
        full((2 * H, H)), full((1, H)), full((H, 1)), full((N, 1)),
    ]
    body = functools.partial(_step_body, Q, B, E, H)
    return pl.pallas_call(
        body,
        grid=(L,),
        in_specs=in_specs,
        out_specs=per_t((L, N, 1)),
        out_shape=jax.ShapeDtypeStruct((L, N, 1), jnp.float32),
        scratch_shapes=[
            pltpu.VMEM((N, H), jnp.float32),
            pltpu.VMEM((E, Q), jnp.float32),
            pltpu.VMEM((E, Q), jnp.float32),
            pltpu.VMEM((Q, E), jnp.float32),
            pltpu.VMEM((Q, E), jnp.float32),
        ],
        compiler_params=pltpu.CompilerParams(
            dimension_semantics=("arbitrary",),
        ),
        interpret=interpret,
    )(*ops)


def _prep_ops(p, r, edge_index, pq_rel, params, q1h):
    B, L = p.shape
    E = edge_index.shape[1]
    Q = pq_rel.shape[1]
    H = params["q_emb"].shape[1]
    N = Q * B
    f32 = jnp.float32

    rep = lambda a: jnp.repeat(a, B, axis=0)

    qcol_all = jnp.transpose(q1h, (1, 2, 0)).reshape(L, N, 1)
    rcol_all = jnp.broadcast_to(
        r.T.astype(f32)[:, None, :], (L, Q, B)).reshape(L, N, 1)

    q_emb = params["q_emb"]
    xq0 = params["xq_emb"][:Q]
    xq1 = params["xq_emb"][Q:]
    q_emb_rep = rep(q_emb)
    u0_rep = rep(xq0 - q_emb)
    du_rep = rep(xq1 - xq0)
    init_rep = rep(params["init_h"])

    ms_, mo_, mi_, gr_, mp_ = (params["mlp_self"], params["mlp_outgo"],
                               params["mlp_income"], params["gru"],
                               params["mlp_pred"])
    ws = jnp.concatenate([mo_["W1"][:2 * H], mi_["W1"][2 * H:]], axis=1)
    wd = jnp.concatenate([mo_["W1"][2 * H:], mi_["W1"][:2 * H]], axis=1)
    bias1 = jnp.concatenate([mo_["b1"], mi_["b1"]]).reshape(1, 2 * H)
    w2blk = jnp.zeros((2 * H, 2 * H), f32)
    w2blk = w2blk.at[:H, :H].set(mo_["W2"]).at[H:, H:].set(mi_["W2"])
    bias2 = jnp.concatenate([mo_["b2"], mi_["b2"]]).reshape(1, 2 * H)

    vw = mp_["W2"] @ params["w_out"]
    c = (mp_["b2"] @ params["w_out"])[0]
    biasq_rep = rep((params["bias"] + c)[:, None])

    ei = edge_index.astype(jnp.int32)
    ei_t = ei.T

    return (qcol_all, rcol_all, ei_t, ei,
            q_emb_rep, u0_rep, du_rep, init_rep,
            ms_["W1"], ms_["b1"].reshape(1, H), ms_["W2"], ms_["b2"].reshape(1, H),
            ws, wd, bias1, w2blk, bias2,
            gr_["Wi"], gr_["bi"].reshape(1, 3 * H), gr_["Wh"], gr_["bh"].reshape(1, 3 * H),
            mp_["W1"], mp_["b1"].reshape(1, H), vw, biasq_rep)


def kernel(p, r, edge_index, pq_rel, params):
    B, L = p.shape
    E = edge_index.shape[1]
    Q = pq_rel.shape[1]
    H = params["q_emb"].shape[1]

    n = B * L
    n_pad = ((n + 255) // 256) * 256
    idx = jnp.concatenate(
        [p.reshape(-1).astype(jnp.int32),
         jnp.zeros((n_pad - n,), jnp.int32)])
    rows = _sc_gather_rows(pq_rel.astype(jnp.float32), idx)
    q1h = rows[:n].reshape(B, L, Q)

    ops = _prep_ops(p, r, edge_index, pq_rel, params, q1h)
    out = _run_scan(L, Q, B, E, H, ops)
    return jnp.transpose(out.reshape(L, Q, B), (2, 0, 1))


# speedup vs baseline: 7.7304x; 1.0369x over previous
import functools

import jax
import jax.numpy as jnp
from jax import lax
from jax.experimental import pallas as pl
from jax.experimental.pallas import tpu as pltpu
from jax.experimental.pallas import tpu_sc as plsc


def _sc_gather_rows(table, idx_pad):
    n_pad = idx_pad.shape[0]
    d = table.shape[1]
    info = plsc.get_sparse_core_info()
    nc, ns = info.num_cores, info.num_subcores
    nw = nc * ns
    b_per_w = n_pad // nw
    mesh = plsc.VectorSubcoreMesh(core_axis_name="c", subcore_axis_name="s")

    @functools.partial(
        pl.kernel,
        mesh=mesh,
        out_type=jax.ShapeDtypeStruct((n_pad, d), jnp.float32),
        scratch_types=[
            pltpu.VMEM((b_per_w,), jnp.int32),
            pltpu.VMEM((b_per_w, d), jnp.float32),
            pltpu.SemaphoreType.DMA,
        ],
    )
    def gather_k(table_hbm, idx_hbm, out_hbm, idx_v, rows_v, sem):
        wid = lax.axis_index("s") * nc + lax.axis_index("c")
        base = wid * b_per_w
        pltpu.sync_copy(idx_hbm.at[pl.ds(base, b_per_w)], idx_v)
        pltpu.async_copy(table_hbm.at[idx_v], rows_v, sem).wait()
        pltpu.sync_copy(rows_v, out_hbm.at[pl.ds(base, b_per_w)])

    return gather_k(table, idx_pad)


def _step_body(Q, B, E, H,
               qcol_ref, rcol_ref, ei_t_ref, ei_ref,
               qemb_ref, u0_ref, du_ref, init_ref,
               w1s_ref, b1s_ref, w2s_ref, b2s_ref,
               ws_ref, wd_ref, bias1_ref, w2blk_ref, bias2_ref,
               wi_ref, bi_ref, wh_ref, bh_ref,
               w1p_ref, b1p_ref, vw_ref, biasq_ref,
               y_ref,
               ht_ref, gsrc_ref, gdst_ref, gdt_ref, gst_ref):
    t = pl.program_id(0)
    N = Q * B

    @pl.when(t == 0)
    def _init():
        ht_ref[...] = init_ref[...]
        src_col = ei_t_ref[:, 0:1]
        dst_col = ei_t_ref[:, 1:2]
        iota_eq = lax.broadcasted_iota(jnp.int32, (E, Q), 1)
        gsrc_ref[...] = (iota_eq == src_col).astype(jnp.bfloat16)
        gdst_ref[...] = (iota_eq == dst_col).astype(jnp.bfloat16)
        src_row = ei_ref[0:1, :]
        dst_row = ei_ref[1:2, :]
        iota_qe = lax.broadcasted_iota(jnp.int32, (Q, E), 0)
        gdt_ref[...] = (iota_qe == dst_row).astype(jnp.bfloat16)
        gst_ref[...] = (iota_qe == src_row).astype(jnp.bfloat16)

    ht = ht_ref[...]
    qcol = qcol_ref[0]
    rcol = rcol_ref[0]

    feat = qemb_ref[...] + qcol * (u0_ref[...] + rcol * du_ref[...])
    m2 = jnp.concatenate([ht, feat], axis=1)

    dot = lambda a, b: jnp.dot(a, b, preferred_element_type=jnp.float32)
    bf = lambda x: x.astype(jnp.bfloat16)

    m2b = bf(m2)

    h1s = jnp.maximum(dot(m2b, w1s_ref[...]) + b1s_ref[...], 0.0)
    m_self = dot(bf(h1s), w2s_ref[...]) + b2s_ref[...]

    ps = dot(m2b, ws_ref[...])
    pd = dot(m2b, wd_ref[...])
    ps_v = bf(ps.reshape(Q, B * 2 * H))
    pd_v = bf(pd.reshape(Q, B * 2 * H))
    pre_v = dot(gsrc_ref[...], ps_v) + dot(gdst_ref[...], pd_v)
    h1m = jnp.maximum(pre_v.reshape(E * B, 2 * H) + bias1_ref[...], 0.0)
    msgs = dot(bf(h1m), w2blk_ref[...]) + bias2_ref[...]

    msgs_v = bf(msgs.reshape(E, B * 2 * H))
    agg_o = dot(gdt_ref[...], msgs_v)
    agg_i = dot(gst_ref[...], msgs_v)
    lane = lax.broadcasted_iota(jnp.int32, (1, B * 2 * H), 1)
    out_half = (lane % (2 * H)) < H
    agg_full = jnp.where(out_half, agg_o, agg_i).reshape(N, 2 * H)
    agg = agg_full[:, :H] + agg_full[:, H:]

    ht_ = m_self + agg

    hcat = jnp.concatenate([ht, ht_], axis=1)
    gi = dot(bf(hcat), wi_ref[...]) + bi_ref[...]
    gh = dot(bf(ht), wh_ref[...]) + bh_ref[...]
    rr = jax.nn.sigmoid(gi[:, :H] + gh[:, :H])
    zz = jax.nn.sigmoid(gi[:, H:2 * H] + gh[:, H:2 * H])
    nn_ = jnp.tanh(gi[:, 2 * H:] + rr * gh[:, 2 * H:])
    hcand = (1.0 - zz) * nn_ + zz * ht
    hnew = qcol * hcand + (1.0 - qcol) * ht
    ht_ref[...] = hnew

    pin = jnp.concatenate([hnew, qemb_ref[...]], axis=1)
    h1p = jnp.maximum(dot(bf(pin), w1p_ref[...]) + b1p_ref[...], 0.0)
    logit = dot(bf(h1p), vw_ref[...]) + biasq_ref[...]
    y_ref[...] = jax.nn.sigmoid(logit).reshape(1, N, 1)


def _run_scan(L, Q, B, E, H, ops, interpret=False):
    N = Q * B
    full = lambda shape: pl.BlockSpec(shape, lambda t: (0,) * len(shape))
    per_t = lambda shape: pl.BlockSpec((1,) + shape[1:], lambda t: (t,) + (0,) * (len(shape) - 1))
    in_specs = [
        per_t((L, N, 1)),
        per_t((L, N, 1)),
        full((E, 2)),
        full((2, E)),
        full((N, H)),
        full((N, H)),
        full((N, H)),
        full((N, H)),
        full((2 * H, H)), full((1, H)), full((H, H)), full((1, H)),
        full((2 * H, 2 * H)), full((2 * H, 2 * H)), full((1, 2 * H)),
        full((2 * H, 2 * H)), full((1, 2 * H)),
        full((2 * H, 3 * H)), full((1, 3 * H)),
        full((H, 3 * H)), full((1, 3 * H)),
        full((2 * H, H)), full((1, H)), full((H, 1)), full((N, 1)),
    ]
    body = functools.partial(_step_body, Q, B, E, H)
    return pl.pallas_call(
        body,
        grid=(L,),
        in_specs=in_specs,
        out_specs=per_t((L, N, 1)),
        out_shape=jax.ShapeDtypeStruct((L, N, 1), jnp.float32),
        scratch_shapes=[
            pltpu.VMEM((N, H), jnp.float32),
            pltpu.VMEM((E, Q), jnp.bfloat16),
            pltpu.VMEM((E, Q), jnp.bfloat16),
            pltpu.VMEM((Q, E), jnp.bfloat16),
            pltpu.VMEM((Q, E), jnp.bfloat16),
        ],
        compiler_params=pltpu.CompilerParams(
            dimension_semantics=("arbitrary",),
        ),
        interpret=interpret,
    )(*ops)


def _prep_ops(p, r, edge_index, pq_rel, params, q1h):
    B, L = p.shape
    E = edge_index.shape[1]
    Q = pq_rel.shape[1]
    H = params["q_emb"].shape[1]
    N = Q * B
    f32 = jnp.float32

    rep = lambda a: jnp.repeat(a, B, axis=0)

    qcol_all = jnp.transpose(q1h, (1, 2, 0)).reshape(L, N, 1)
    rcol_all = jnp.broadcast_to(
        r.T.astype(f32)[:, None, :], (L, Q, B)).reshape(L, N, 1)

    q_emb = params["q_emb"]
    xq0 = params["xq_emb"][:Q]
    xq1 = params["xq_emb"][Q:]
    q_emb_rep = rep(q_emb)
    u0_rep = rep(xq0 - q_emb)
    du_rep = rep(xq1 - xq0)
    init_rep = rep(params["init_h"])

    ms_, mo_, mi_, gr_, mp_ = (params["mlp_self"], params["mlp_outgo"],
                               params["mlp_income"], params["gru"],
                               params["mlp_pred"])
    ws = jnp.concatenate([mo_["W1"][:2 * H], mi_["W1"][2 * H:]], axis=1)
    wd = jnp.concatenate([mo_["W1"][2 * H:], mi_["W1"][:2 * H]], axis=1)
    bias1 = jnp.concatenate([mo_["b1"], mi_["b1"]]).reshape(1, 2 * H)
    w2blk = jnp.zeros((2 * H, 2 * H), f32)
    w2blk = w2blk.at[:H, :H].set(mo_["W2"]).at[H:, H:].set(mi_["W2"])
    bias2 = jnp.concatenate([mo_["b2"], mi_["b2"]]).reshape(1, 2 * H)

    vw = mp_["W2"] @ params["w_out"]
    c = (mp_["b2"] @ params["w_out"])[0]
    biasq_rep = rep((params["bias"] + c)[:, None])

    ei = edge_index.astype(jnp.int32)
    ei_t = ei.T

    b16 = lambda a: a.astype(jnp.bfloat16)
    return (qcol_all, rcol_all, ei_t, ei,
            q_emb_rep, u0_rep, du_rep, init_rep,
            b16(ms_["W1"]), ms_["b1"].reshape(1, H), b16(ms_["W2"]), ms_["b2"].reshape(1, H),
            b16(ws), b16(wd), bias1, b16(w2blk), bias2,
            b16(gr_["Wi"]), gr_["bi"].reshape(1, 3 * H), b16(gr_["Wh"]), gr_["bh"].reshape(1, 3 * H),
            b16(mp_["W1"]), mp_["b1"].reshape(1, H), b16(vw), biasq_rep)


def kernel(p, r, edge_index, pq_rel, params):
    B, L = p.shape
    E = edge_index.shape[1]
    Q = pq_rel.shape[1]
    H = params["q_emb"].shape[1]

    n = B * L
    n_pad = ((n + 255) // 256) * 256
    idx = jnp.concatenate(
        [p.reshape(-1).astype(jnp.int32),
         jnp.zeros((n_pad - n,), jnp.int32)])
    rows = _sc_gather_rows(pq_rel.astype(jnp.float32), idx)
    q1h = rows[:n].reshape(B, L, Q)

    ops = _prep_ops(p, r, edge_index, pq_rel, params, q1h)
    out = _run_scan(L, Q, B, E, H, ops)
    return jnp.transpose(out.reshape(L, Q, B), (2, 0, 1))


# speedup vs baseline: 8.4058x; 1.0874x over previous
import functools

import jax
import jax.numpy as jnp
from jax import lax
from jax.experimental import pallas as pl
from jax.experimental.pallas import tpu as pltpu
from jax.experimental.pallas import tpu_sc as plsc


def _sc_gather_rows(table, idx_pad):
    n_pad = idx_pad.shape[0]
    d = table.shape[1]
    info = plsc.get_sparse_core_info()
    nc, ns = info.num_cores, info.num_subcores
    nw = nc * ns
    b_per_w = n_pad // nw
    mesh = plsc.VectorSubcoreMesh(core_axis_name="c", subcore_axis_name="s")

    @functools.partial(
        pl.kernel,
        mesh=mesh,
        out_type=jax.ShapeDtypeStruct((n_pad, d), jnp.float32),
        scratch_types=[
            pltpu.VMEM((b_per_w,), jnp.int32),
            pltpu.VMEM((b_per_w, d), jnp.float32),
            pltpu.SemaphoreType.DMA,
        ],
    )
    def gather_k(table_hbm, idx_hbm, out_hbm, idx_v, rows_v, sem):
        wid = lax.axis_index("s") * nc + lax.axis_index("c")
        base = wid * b_per_w
        pltpu.sync_copy(idx_hbm.at[pl.ds(base, b_per_w)], idx_v)
        pltpu.async_copy(table_hbm.at[idx_v], rows_v, sem).wait()
        pltpu.sync_copy(rows_v, out_hbm.at[pl.ds(base, b_per_w)])

    return gather_k(table, idx_pad)


def _step_body(Q, B, E, H,
               qcol_ref, rcol_ref, ei_t_ref, ei_ref,
               qemb_ref, u0_ref, du_ref, init_ref,
               w1s_ref, b1s_ref, w2s_ref, b2s_ref,
               ws_ref, wd_ref, bias1_ref, w2blk_ref, b2o_ref, b2i_ref,
               wi_ref, bi_ref, wh_ref, bh_ref,
               w1p_ref, b1p_ref, vw_ref, biasq_ref,
               y_ref,
               ht_ref, gcomb_ref, gdt_ref, gst_ref, bagg_ref):
    t = pl.program_id(0)
    N = Q * B

    @pl.when(t == 0)
    def _init():
        ht_ref[...] = init_ref[...]
        src_col = ei_t_ref[:, 0:1]
        dst_col = ei_t_ref[:, 1:2]
        iota_eq = lax.broadcasted_iota(jnp.int32, (E, 2 * Q), 1)
        gcomb_ref[...] = jnp.logical_or(
            iota_eq == src_col, iota_eq == (dst_col + Q)).astype(jnp.bfloat16)
        src_row = ei_ref[0:1, :]
        dst_row = ei_ref[1:2, :]
        iota_qe = lax.broadcasted_iota(jnp.int32, (Q, E), 0)
        gdt = (iota_qe == dst_row).astype(jnp.float32)
        gst = (iota_qe == src_row).astype(jnp.float32)
        gdt_ref[...] = gdt.astype(jnp.bfloat16)
        gst_ref[...] = gst.astype(jnp.bfloat16)
        deg_d = jnp.sum(gdt, axis=1, keepdims=True)
        deg_s = jnp.sum(gst, axis=1, keepdims=True)
        bnode = (deg_d * b2o_ref[...] + deg_s * b2i_ref[...]
                 + b2s_ref[...])
        bagg_ref[...] = jnp.broadcast_to(
            bnode[:, None, :], (Q, B, H)).reshape(N, H)

    ht = ht_ref[...]
    qcol = qcol_ref[0]
    rcol = rcol_ref[0]

    feat = qemb_ref[...] + qcol * (u0_ref[...] + rcol * du_ref[...])
    m2 = jnp.concatenate([ht, feat], axis=1)

    dot = lambda a, b: jnp.dot(a, b, preferred_element_type=jnp.float32)
    bf = lambda x: x.astype(jnp.bfloat16)
    dotb = lambda a, b: bf(jnp.dot(a, b, preferred_element_type=jnp.float32))

    m2b = bf(m2)

    h1s = jnp.maximum(dot(m2b, w1s_ref[...]) + b1s_ref[...], 0.0)
    m_self = dot(bf(h1s), w2s_ref[...])

    ps = dotb(m2b, ws_ref[...]) + bias1_ref[...]
    pd = dotb(m2b, wd_ref[...])
    psd = jnp.concatenate(
        [ps.reshape(Q, B * 2 * H), pd.reshape(Q, B * 2 * H)], axis=0)
    pre_v = dotb(gcomb_ref[...], psd)
    h1m = jnp.maximum(pre_v, 0).reshape(E * B, 2 * H)
    msgs = dotb(h1m, w2blk_ref[...])

    msgs_v = msgs.reshape(E, B * 2 * H)
    agg_o = dot(gdt_ref[...], msgs_v)
    agg_i = dot(gst_ref[...], msgs_v)
    lane = lax.broadcasted_iota(jnp.int32, (1, B * 2 * H), 1)
    out_half = (lane % (2 * H)) < H
    agg_full = jnp.where(out_half, agg_o, agg_i).reshape(N, 2 * H)
    agg = agg_full[:, :H] + agg_full[:, H:]

    ht_ = m_self + agg + bagg_ref[...]

    hcat = jnp.concatenate([ht, ht_], axis=1)
    gi = dot(bf(hcat), wi_ref[...]) + bi_ref[...]
    gh = dot(bf(ht), wh_ref[...]) + bh_ref[...]
    rr = jax.nn.sigmoid(gi[:, :H] + gh[:, :H])
    zz = jax.nn.sigmoid(gi[:, H:2 * H] + gh[:, H:2 * H])
    nn_ = jnp.tanh(gi[:, 2 * H:] + rr * gh[:, 2 * H:])
    hcand = (1.0 - zz) * nn_ + zz * ht
    hnew = qcol * hcand + (1.0 - qcol) * ht
    ht_ref[...] = hnew

    pin = jnp.concatenate([hnew, qemb_ref[...]], axis=1)
    h1p = jnp.maximum(dot(bf(pin), w1p_ref[...]) + b1p_ref[...], 0.0)
    logit = dot(bf(h1p), vw_ref[...]) + biasq_ref[...]
    y_ref[...] = jax.nn.sigmoid(logit).reshape(1, N, 1)


def _run_scan(L, Q, B, E, H, ops, interpret=False):
    N = Q * B
    full = lambda shape: pl.BlockSpec(shape, lambda t: (0,) * len(shape))
    per_t = lambda shape: pl.BlockSpec((1,) + shape[1:], lambda t: (t,) + (0,) * (len(shape) - 1))
    in_specs = [
        per_t((L, N, 1)),
        per_t((L, N, 1)),
        full((E, 2)),
        full((2, E)),
        full((N, H)),
        full((N, H)),
        full((N, H)),
        full((N, H)),
        full((2 * H, H)), full((1, H)), full((H, H)), full((1, H)),
        full((2 * H, 2 * H)), full((2 * H, 2 * H)), full((1, 2 * H)),
        full((2 * H, 2 * H)), full((1, H)), full((1, H)),
        full((2 * H, 3 * H)), full((1, 3 * H)),
        full((H, 3 * H)), full((1, 3 * H)),
        full((2 * H, H)), full((1, H)), full((H, 1)), full((N, 1)),
    ]
    body = functools.partial(_step_body, Q, B, E, H)
    return pl.pallas_call(
        body,
        grid=(L,),
        in_specs=in_specs,
        out_specs=per_t((L, N, 1)),
        out_shape=jax.ShapeDtypeStruct((L, N, 1), jnp.float32),
        scratch_shapes=[
            pltpu.VMEM((N, H), jnp.float32),
            pltpu.VMEM((E, 2 * Q), jnp.bfloat16),
            pltpu.VMEM((Q, E), jnp.bfloat16),
            pltpu.VMEM((Q, E), jnp.bfloat16),
            pltpu.VMEM((N, H), jnp.float32),
        ],
        compiler_params=pltpu.CompilerParams(
            dimension_semantics=("arbitrary",),
        ),
        interpret=interpret,
    )(*ops)


def _prep_ops(p, r, edge_index, pq_rel, params, q1h):
    B, L = p.shape
    E = edge_index.shape[1]
    Q = pq_rel.shape[1]
    H = params["q_emb"].shape[1]
    N = Q * B
    f32 = jnp.float32

    rep = lambda a: jnp.repeat(a, B, axis=0)

    qcol_all = jnp.transpose(q1h, (1, 2, 0)).reshape(L, N, 1)
    rcol_all = jnp.broadcast_to(
        r.T.astype(f32)[:, None, :], (L, Q, B)).reshape(L, N, 1)

    q_emb = params["q_emb"]
    xq0 = params["xq_emb"][:Q]
    xq1 = params["xq_emb"][Q:]
    q_emb_rep = rep(q_emb)
    u0_rep = rep(xq0 - q_emb)
    du_rep = rep(xq1 - xq0)
    init_rep = rep(params["init_h"])

    ms_, mo_, mi_, gr_, mp_ = (params["mlp_self"], params["mlp_outgo"],
                               params["mlp_income"], params["gru"],
                               params["mlp_pred"])
    ws = jnp.concatenate([mo_["W1"][:2 * H], mi_["W1"][2 * H:]], axis=1)
    wd = jnp.concatenate([mo_["W1"][2 * H:], mi_["W1"][:2 * H]], axis=1)
    bias1 = jnp.concatenate([mo_["b1"], mi_["b1"]]).reshape(1, 2 * H)
    w2blk = jnp.zeros((2 * H, 2 * H), f32)
    w2blk = w2blk.at[:H, :H].set(mo_["W2"]).at[H:, H:].set(mi_["W2"])

    vw = mp_["W2"] @ params["w_out"]
    c = (mp_["b2"] @ params["w_out"])[0]
    biasq_rep = rep((params["bias"] + c)[:, None])

    ei = edge_index.astype(jnp.int32)
    ei_t = ei.T

    b16 = lambda a: a.astype(jnp.bfloat16)
    return (qcol_all, rcol_all, ei_t, ei,
            q_emb_rep, u0_rep, du_rep, init_rep,
            b16(ms_["W1"]), ms_["b1"].reshape(1, H), b16(ms_["W2"]), ms_["b2"].reshape(1, H),
            b16(ws), b16(wd), b16(bias1), b16(w2blk),
            mo_["b2"].reshape(1, H), mi_["b2"].reshape(1, H),
            b16(gr_["Wi"]), gr_["bi"].reshape(1, 3 * H), b16(gr_["Wh"]), gr_["bh"].reshape(1, 3 * H),
            b16(mp_["W1"]), mp_["b1"].reshape(1, H), b16(vw), biasq_rep)


def kernel(p, r, edge_index, pq_rel, params):
    B, L = p.shape
    E = edge_index.shape[1]
    Q = pq_rel.shape[1]
    H = params["q_emb"].shape[1]

    n = B * L
    n_pad = ((n + 255) // 256) * 256
    idx = jnp.concatenate(
        [p.reshape(-1).astype(jnp.int32),
         jnp.zeros((n_pad - n,), jnp.int32)])
    rows = _sc_gather_rows(pq_rel.astype(jnp.float32), idx)
    q1h = rows[:n].reshape(B, L, Q)

    ops = _prep_ops(p, r, edge_index, pq_rel, params, q1h)
    out = _run_scan(L, Q, B, E, H, ops)
    return jnp.transpose(out.reshape(L, Q, B), (2, 0, 1))


# speedup vs baseline: 11.9395x; 1.4204x over previous
import functools

import jax
import jax.numpy as jnp
from jax import lax
from jax.experimental import pallas as pl
from jax.experimental.pallas import tpu as pltpu
from jax.experimental.pallas import tpu_sc as plsc


def _sc_gather_rows(table, idx_pad):
    n_pad = idx_pad.shape[0]
    d = table.shape[1]
    info = plsc.get_sparse_core_info()
    nc, ns = info.num_cores, info.num_subcores
    nw = nc * ns
    b_per_w = n_pad // nw
    mesh = plsc.VectorSubcoreMesh(core_axis_name="c", subcore_axis_name="s")

    @functools.partial(
        pl.kernel,
        mesh=mesh,
        out_type=jax.ShapeDtypeStruct((n_pad, d), jnp.float32),
        scratch_types=[
            pltpu.VMEM((b_per_w,), jnp.int32),
            pltpu.VMEM((b_per_w, d), jnp.float32),
            pltpu.SemaphoreType.DMA,
        ],
    )
    def gather_k(table_hbm, idx_hbm, out_hbm, idx_v, rows_v, sem):
        wid = lax.axis_index("s") * nc + lax.axis_index("c")
        base = wid * b_per_w
        pltpu.sync_copy(idx_hbm.at[pl.ds(base, b_per_w)], idx_v)
        pltpu.async_copy(table_hbm.at[idx_v], rows_v, sem).wait()
        pltpu.sync_copy(rows_v, out_hbm.at[pl.ds(base, b_per_w)])

    return gather_k(table, idx_pad)


def _step_body(Q, B, E, H,
               q1ht_ref, rt_ref, ei_t_ref, ei_ref,
               qemb_ref, xq_ref, init_ref, bias_ref,
               w1s_ref, b1s_ref, w2s_ref, b2s_ref,
               w1o_ref, b1o_ref, w2o_ref, b2o_ref,
               w1i_ref, b1i_ref, w2i_ref, b2i_ref,
               wi_ref, bi_ref, wh_ref, bh_ref,
               w1p_ref, b1p_ref, w2p_ref, b2p_ref, wout_ref,
               y_ref,
               ht_ref, gcomb_ref, gdt_ref, gst_ref, bagg_ref,
               qembr_ref, u0r_ref, dur_ref, biasq_ref,
               wsb_ref, wdb_ref, b1b_ref, w2blk_ref,
               w1sb_ref, w2sb_ref, wib_ref, whb_ref, w1pb_ref, vwb_ref):
    t = pl.program_id(0)
    N = Q * B

    dot = lambda a, b: jnp.dot(a, b, preferred_element_type=jnp.float32)
    bf = lambda x: x.astype(jnp.bfloat16)
    dotb = lambda a, b: bf(jnp.dot(a, b, preferred_element_type=jnp.float32))
    rep = lambda x: jnp.broadcast_to(x[:, None, :], (Q, B, x.shape[-1])
                                     ).reshape(N, x.shape[-1])

    @pl.when(t == 0)
    def _init():
        ht_ref[...] = rep(init_ref[...])
        src_col = ei_t_ref[:, 0:1]
        dst_col = ei_t_ref[:, 1:2]
        iota_eq = lax.broadcasted_iota(jnp.int32, (E, 2 * Q), 1)
        gcomb_ref[...] = jnp.logical_or(
            iota_eq == src_col, iota_eq == (dst_col + Q)).astype(jnp.bfloat16)
        src_row = ei_ref[0:1, :]
        dst_row = ei_ref[1:2, :]
        iota_qe = lax.broadcasted_iota(jnp.int32, (Q, E), 0)
        gdt = (iota_qe == dst_row).astype(jnp.float32)
        gst = (iota_qe == src_row).astype(jnp.float32)
        gdt_ref[...] = gdt.astype(jnp.bfloat16)
        gst_ref[...] = gst.astype(jnp.bfloat16)
        deg_d = jnp.sum(gdt, axis=1, keepdims=True)
        deg_s = jnp.sum(gst, axis=1, keepdims=True)
        bnode = (deg_d * b2o_ref[...] + deg_s * b2i_ref[...]
                 + b2s_ref[...])
        bagg_ref[...] = rep(bnode)
        qemb = qemb_ref[...]
        xq0 = xq_ref[:Q, :]
        xq1 = xq_ref[Q:, :]
        qembr_ref[...] = bf(rep(qemb))
        u0r_ref[...] = bf(rep(xq0 - qemb))
        dur_ref[...] = bf(rep(xq1 - xq0))
        biasq_ref[...] = rep(bias_ref[...]) + dot(b2p_ref[...], wout_ref[...])
        w1o = w1o_ref[...]
        w1i = w1i_ref[...]
        wsb_ref[...] = bf(jnp.concatenate([w1o[:2 * H], w1i[2 * H:]], axis=1))
        wdb_ref[...] = bf(jnp.concatenate([w1o[2 * H:], w1i[:2 * H]], axis=1))
        b1b_ref[...] = bf(jnp.concatenate([b1o_ref[...], b1i_ref[...]], axis=1))
        z = jnp.zeros((H, H), jnp.float32)
        w2blk_ref[...] = bf(jnp.concatenate([
            jnp.concatenate([w2o_ref[...], z], axis=1),
            jnp.concatenate([z, w2i_ref[...]], axis=1)], axis=0))
        w1sb_ref[...] = bf(w1s_ref[...])
        w2sb_ref[...] = bf(w2s_ref[...])
        wib_ref[...] = bf(wi_ref[...])
        whb_ref[...] = bf(wh_ref[...])
        w1pb_ref[...] = bf(w1p_ref[...])
        vwb_ref[...] = bf(dot(w2p_ref[...], wout_ref[...]))

    ht = ht_ref[...]
    qembr = qembr_ref[...]

    qm = q1ht_ref[0]
    qv = jnp.broadcast_to(qm[:, :, None], (Q, B, H)).reshape(N, H)
    qvb = bf(qv)
    rv = rt_ref[0]
    rvb = bf(jnp.broadcast_to(rv[:, :, None], (Q, B, H)).reshape(N, H))

    featb = qembr + qvb * (u0r_ref[...] + rvb * dur_ref[...])
    m2b = jnp.concatenate([bf(ht), featb], axis=1)

    h1s = jnp.maximum(dot(m2b, w1sb_ref[...]) + b1s_ref[...], 0.0)
    m_self = dot(bf(h1s), w2sb_ref[...])

    ps = dotb(m2b, wsb_ref[...]) + b1b_ref[...]
    pd = dotb(m2b, wdb_ref[...])
    psd = jnp.concatenate(
        [ps.reshape(Q, B * 2 * H), pd.reshape(Q, B * 2 * H)], axis=0)
    pre_v = dotb(gcomb_ref[...], psd)
    h1m = jnp.maximum(pre_v, 0).reshape(E * B, 2 * H)
    msgs = dotb(h1m, w2blk_ref[...])

    msgs_v = msgs.reshape(E, B * 2 * H)
    agg_o = dot(gdt_ref[...], msgs_v)
    agg_i = dot(gst_ref[...], msgs_v)
    lane = lax.broadcasted_iota(jnp.int32, (1, B * 2 * H), 1)
    out_half = (lane % (2 * H)) < H
    agg_full = jnp.where(out_half, agg_o, agg_i).reshape(N, 2 * H)
    agg = agg_full[:, :H] + agg_full[:, H:]

    ht_ = m_self + agg + bagg_ref[...]

    hcat = jnp.concatenate([bf(ht), bf(ht_)], axis=1)
    gi = dot(hcat, wib_ref[...]) + bi_ref[...]
    gh = dot(bf(ht), whb_ref[...]) + bh_ref[...]
    rr = jax.nn.sigmoid(gi[:, :H] + gh[:, :H])
    zz = jax.nn.sigmoid(gi[:, H:2 * H] + gh[:, H:2 * H])
    nn_ = jnp.tanh(gi[:, 2 * H:] + rr * gh[:, 2 * H:])
    hcand = (1.0 - zz) * nn_ + zz * ht
    hnew = ht + qv * (hcand - ht)
    ht_ref[...] = hnew

    pin = jnp.concatenate([bf(hnew), qembr], axis=1)
    h1p = jnp.maximum(dot(pin, w1pb_ref[...]) + b1p_ref[...], 0.0)
    logit = dot(bf(h1p), vwb_ref[...]) + biasq_ref[...]
    y_ref[...] = jax.nn.sigmoid(logit).reshape(1, N, 1)


def _run_scan(L, Q, B, E, H, ops, interpret=False):
    N = Q * B
    bf16 = jnp.bfloat16
    f32 = jnp.float32
    full = lambda shape: pl.BlockSpec(shape, lambda t: (0,) * len(shape))
    per_t = lambda shape: pl.BlockSpec((1,) + shape[1:], lambda t: (t,) + (0,) * (len(shape) - 1))
    in_specs = [
        per_t((L, Q, B)),
        per_t((L, 1, B)),
        full((E, 2)),
        full((2, E)),
        full((Q, H)), full((2 * Q, H)), full((Q, H)), full((Q, 1)),
        full((2 * H, H)), full((1, H)), full((H, H)), full((1, H)),
        full((4 * H, H)), full((1, H)), full((H, H)), full((1, H)),
        full((4 * H, H)), full((1, H)), full((H, H)), full((1, H)),
        full((2 * H, 3 * H)), full((1, 3 * H)), full((H, 3 * H)), full((1, 3 * H)),
        full((2 * H, H)), full((1, H)), full((H, H)), full((1, H)), full((H, 1)),
    ]
    body = functools.partial(_step_body, Q, B, E, H)
    return pl.pallas_call(
        body,
        grid=(L,),
        in_specs=in_specs,
        out_specs=per_t((L, N, 1)),
        out_shape=jax.ShapeDtypeStruct((L, N, 1), f32),
        scratch_shapes=[
            pltpu.VMEM((N, H), f32),
            pltpu.VMEM((E, 2 * Q), bf16),
            pltpu.VMEM((Q, E), bf16),
            pltpu.VMEM((Q, E), bf16),
            pltpu.VMEM((N, H), f32),
            pltpu.VMEM((N, H), bf16),
            pltpu.VMEM((N, H), bf16),
            pltpu.VMEM((N, H), bf16),
            pltpu.VMEM((N, 1), f32),
            pltpu.VMEM((2 * H, 2 * H), bf16),
            pltpu.VMEM((2 * H, 2 * H), bf16),
            pltpu.VMEM((1, 2 * H), bf16),
            pltpu.VMEM((2 * H, 2 * H), bf16),
            pltpu.VMEM((2 * H, H), bf16),
            pltpu.VMEM((H, H), bf16),
            pltpu.VMEM((2 * H, 3 * H), bf16),
            pltpu.VMEM((H, 3 * H), bf16),
            pltpu.VMEM((2 * H, H), bf16),
            pltpu.VMEM((H, 1), bf16),
        ],
        compiler_params=pltpu.CompilerParams(
            dimension_semantics=("arbitrary",),
        ),
        interpret=interpret,
    )(*ops)


def kernel(p, r, edge_index, pq_rel, params):
    B, L = p.shape
    E = edge_index.shape[1]
    Q = pq_rel.shape[1]
    H = params["q_emb"].shape[1]
    f32 = jnp.float32

    n = B * L
    n_pad = ((n + 255) // 256) * 256
    idx = jnp.concatenate(
        [p.reshape(-1).astype(jnp.int32),
         jnp.zeros((n_pad - n,), jnp.int32)])
    rows = _sc_gather_rows(pq_rel.astype(f32), idx)
    q1ht = jnp.transpose(rows[:n].reshape(B, L, Q), (1, 2, 0))

    rt = r.T.astype(f32).reshape(L, 1, B)
    ei = edge_index.astype(jnp.int32)
    ei_t = ei.T
    pr = params
    row = lambda a: a.reshape(1, -1)
    ops = (q1ht, rt, ei_t, ei,
           pr["q_emb"], pr["xq_emb"], pr["init_h"], pr["bias"].reshape(Q, 1),
           pr["mlp_self"]["W1"], row(pr["mlp_self"]["b1"]),
           pr["mlp_self"]["W2"], row(pr["mlp_self"]["b2"]),
           pr["mlp_outgo"]["W1"], row(pr["mlp_outgo"]["b1"]),
           pr["mlp_outgo"]["W2"], row(pr["mlp_outgo"]["b2"]),
           pr["mlp_income"]["W1"], row(pr["mlp_income"]["b1"]),
           pr["mlp_income"]["W2"], row(pr["mlp_income"]["b2"]),
           pr["gru"]["Wi"], row(pr["gru"]["bi"]),
           pr["gru"]["Wh"], row(pr["gru"]["bh"]),
           pr["mlp_pred"]["W1"], row(pr["mlp_pred"]["b1"]),
           pr["mlp_pred"]["W2"], row(pr["mlp_pred"]["b2"]), pr["w_out"])
    out = _run_scan(L, Q, B, E, H, ops)
    return jnp.transpose(out.reshape(L, Q, B), (2, 0, 1))


# speedup vs baseline: 15.5937x; 1.3061x over previous
import functools

import jax
import jax.numpy as jnp
from jax import lax
from jax.experimental import pallas as pl
from jax.experimental.pallas import tpu as pltpu
from jax.experimental.pallas import tpu_sc as plsc


def _sc_gather_rows(table, idx_pad):
    n_pad = idx_pad.shape[0]
    d = table.shape[1]
    info = plsc.get_sparse_core_info()
    nc, ns = info.num_cores, info.num_subcores
    nw = nc * ns
    b_per_w = n_pad // nw
    mesh = plsc.VectorSubcoreMesh(core_axis_name="c", subcore_axis_name="s")

    @functools.partial(
        pl.kernel,
        mesh=mesh,
        out_type=jax.ShapeDtypeStruct((n_pad, d), jnp.float32),
        scratch_types=[
            pltpu.VMEM((b_per_w,), jnp.int32),
            pltpu.VMEM((b_per_w, d), jnp.float32),
            pltpu.SemaphoreType.DMA,
        ],
    )
    def gather_k(table_hbm, idx_hbm, out_hbm, idx_v, rows_v, sem):
        wid = lax.axis_index("s") * nc + lax.axis_index("c")
        base = wid * b_per_w
        pltpu.sync_copy(idx_hbm.at[pl.ds(base, b_per_w)], idx_v)
        pltpu.async_copy(table_hbm.at[idx_v], rows_v, sem).wait()
        pltpu.sync_copy(rows_v, out_hbm.at[pl.ds(base, b_per_w)])

    return gather_k(table, idx_pad)


def _step_body(Q, B, E, H,
               q1ht_ref, rt_ref, ei_t_ref, ei_ref,
               qemb_ref, xq_ref, init_ref, bias_ref,
               w1s_ref, b1s_ref, w2s_ref, b2s_ref,
               w1o_ref, b1o_ref, w2o_ref, b2o_ref,
               w1i_ref, b1i_ref, w2i_ref, b2i_ref,
               wi_ref, bi_ref, wh_ref, bh_ref,
               w1p_ref, b1p_ref, w2p_ref, b2p_ref, wout_ref,
               y_ref,
               ht_ref, gcomb_ref, gdt_ref, gst_ref, bagg_ref,
               qembr_ref, u0r_ref, dur_ref, biasq_ref,
               wallb_ref, b1b_ref, w2sb_ref, w2stk_ref,
               wgrzb_ref, brz_ref, winb_ref, whnb_ref,
               w1ptb_ref, qep_ref, vwb_ref):
    t = pl.program_id(0)
    N = Q * B

    dot = lambda a, b: jnp.dot(a, b, preferred_element_type=jnp.float32)
    bf = lambda x: x.astype(jnp.bfloat16)
    rep = lambda x: jnp.broadcast_to(x[:, None, :], (Q, B, x.shape[-1])
                                     ).reshape(N, x.shape[-1])

    @pl.when(t == 0)
    def _init():
        ht_ref[...] = rep(init_ref[...])
        src_col = ei_t_ref[:, 0:1]
        dst_col = ei_t_ref[:, 1:2]
        iota_eq = lax.broadcasted_iota(jnp.int32, (E, 2 * Q), 1)
        gcomb_ref[...] = jnp.logical_or(
            iota_eq == src_col, iota_eq == (dst_col + Q)).astype(jnp.bfloat16)
        src_row = ei_ref[0:1, :]
        dst_row = ei_ref[1:2, :]
        iota_qe = lax.broadcasted_iota(jnp.int32, (Q, E), 0)
        gdt = (iota_qe == dst_row).astype(jnp.float32)
        gst = (iota_qe == src_row).astype(jnp.float32)
        gdt_ref[...] = gdt.astype(jnp.bfloat16)
        gst_ref[...] = gst.astype(jnp.bfloat16)
        deg_d = jnp.sum(gdt, axis=1, keepdims=True)
        deg_s = jnp.sum(gst, axis=1, keepdims=True)
        bnode = (deg_d * b2o_ref[...] + deg_s * b2i_ref[...]
                 + b2s_ref[...])
        bagg_ref[...] = rep(bnode)
        qemb = qemb_ref[...]
        xq0 = xq_ref[:Q, :]
        xq1 = xq_ref[Q:, :]
        f0 = rep(qemb)
        qembr_ref[...] = bf(f0)
        u0r_ref[...] = bf(rep(xq0 - qemb))
        dur_ref[...] = bf(rep(xq1 - xq0))
        biasq_ref[...] = rep(bias_ref[...]) + dot(b2p_ref[...], wout_ref[...])
        w1o = w1o_ref[...]
        w1i = w1i_ref[...]
        wsb = jnp.concatenate([w1o[:2 * H], w1i[2 * H:]], axis=1)
        wdb = jnp.concatenate([w1o[2 * H:], w1i[:2 * H]], axis=1)
        wallb_ref[...] = bf(jnp.concatenate([wsb, wdb, w1s_ref[...]], axis=1))
        b1b_ref[...] = jnp.concatenate([b1o_ref[...], b1i_ref[...]], axis=1)
        w2sb_ref[...] = bf(w2s_ref[...])
        w2stk_ref[...] = bf(jnp.concatenate(
            [w2o_ref[...], w2i_ref[...]], axis=0))
        wi = wi_ref[...]
        wh = wh_ref[...]
        wgrzb_ref[...] = bf(wi[:, :2 * H] + jnp.concatenate(
            [wh[:, :2 * H], jnp.zeros((H, 2 * H), jnp.float32)], axis=0))
        brz_ref[...] = bi_ref[...][:, :2 * H] + bh_ref[...][:, :2 * H]
        winb_ref[...] = bf(wi[:, 2 * H:])
        whnb_ref[...] = bf(wh[:, 2 * H:])
        w1p = w1p_ref[...]
        w1ptb_ref[...] = bf(w1p[:H])
        qep_ref[...] = bf(dot(f0, w1p[H:]) + b1p_ref[...])
        vwb_ref[...] = bf(dot(w2p_ref[...], wout_ref[...]))

    ht = ht_ref[...]
    htb = bf(ht)

    qm = q1ht_ref[0]
    qv = jnp.broadcast_to(qm[:, :, None], (Q, B, H)).reshape(N, H)
    qvb = bf(qv)
    rv = rt_ref[0]
    rvb = bf(jnp.broadcast_to(rv[:, :, None], (Q, B, H)).reshape(N, H))

    featb = qembr_ref[...] + qvb * (u0r_ref[...] + rvb * dur_ref[...])
    m2b = jnp.concatenate([htb, featb], axis=1)

    proj = dot(m2b, wallb_ref[...])

    h1s = jnp.maximum(proj[:, 4 * H:] + b1s_ref[...], 0.0)
    m_self = dot(bf(h1s), w2sb_ref[...])

    ps = bf(proj[:, :2 * H] + b1b_ref[...])
    pd = bf(proj[:, 2 * H:4 * H])
    psd = jnp.concatenate(
        [ps.reshape(Q, B * 2 * H), pd.reshape(Q, B * 2 * H)], axis=0)
    pre_v = dot(gcomb_ref[...], psd)
    h1v = bf(jnp.maximum(pre_v, 0.0))

    a_d = dot(gdt_ref[...], h1v)
    a_s = dot(gst_ref[...], h1v)
    lane = lax.broadcasted_iota(jnp.int32, (1, B * 2 * H), 1)
    out_half = (lane % (2 * H)) < H
    comb = bf(jnp.where(out_half, a_d, a_s)).reshape(N, 2 * H)
    agg = dot(comb, w2stk_ref[...])

    ht_ = m_self + agg + bagg_ref[...]

    hcat = jnp.concatenate([htb, bf(ht_)], axis=1)
    grz = dot(hcat, wgrzb_ref[...]) + brz_ref[...]
    rr = jax.nn.sigmoid(grz[:, :H])
    zz = jax.nn.sigmoid(grz[:, H:])
    gin = dot(hcat, winb_ref[...]) + bi_ref[...][:, 2 * H:]
    ghn = dot(htb, whnb_ref[...]) + bh_ref[...][:, 2 * H:]
    nn_ = jnp.tanh(gin + rr * ghn)
    hcand = (1.0 - zz) * nn_ + zz * ht
    hnew = ht + qv * (hcand - ht)
    ht_ref[...] = hnew

    h1p = jnp.maximum(dot(bf(hnew), w1ptb_ref[...]) + qep_ref[...], 0.0)
    logit = dot(bf(h1p), vwb_ref[...]) + biasq_ref[...]
    y_ref[...] = jax.nn.sigmoid(logit).reshape(1, N, 1)


def _run_scan(L, Q, B, E, H, ops, interpret=False):
    N = Q * B
    bf16 = jnp.bfloat16
    f32 = jnp.float32
    full = lambda shape: pl.BlockSpec(shape, lambda t: (0,) * len(shape))
    per_t = lambda shape: pl.BlockSpec((1,) + shape[1:], lambda t: (t,) + (0,) * (len(shape) - 1))
    in_specs = [
        per_t((L, Q, B)),
        per_t((L, 1, B)),
        full((E, 2)),
        full((2, E)),
        full((Q, H)), full((2 * Q, H)), full((Q, H)), full((Q, 1)),
        full((2 * H, H)), full((1, H)), full((H, H)), full((1, H)),
        full((4 * H, H)), full((1, H)), full((H, H)), full((1, H)),
        full((4 * H, H)), full((1, H)), full((H, H)), full((1, H)),
        full((2 * H, 3 * H)), full((1, 3 * H)), full((H, 3 * H)), full((1, 3 * H)),
        full((2 * H, H)), full((1, H)), full((H, H)), full((1, H)), full((H, 1)),
    ]
    body = functools.partial(_step_body, Q, B, E, H)
    return pl.pallas_call(
        body,
        grid=(L,),
        in_specs=in_specs,
        out_specs=per_t((L, N, 1)),
        out_shape=jax.ShapeDtypeStruct((L, N, 1), f32),
        scratch_shapes=[
            pltpu.VMEM((N, H), f32),
            pltpu.VMEM((E, 2 * Q), bf16),
            pltpu.VMEM((Q, E), bf16),
            pltpu.VMEM((Q, E), bf16),
            pltpu.VMEM((N, H), f32),
            pltpu.VMEM((N, H), bf16),
            pltpu.VMEM((N, H), bf16),
            pltpu.VMEM((N, H), bf16),
            pltpu.VMEM((N, 1), f32),
            pltpu.VMEM((2 * H, 5 * H), bf16),
            pltpu.VMEM((1, 2 * H), f32),
            pltpu.VMEM((H, H), bf16),
            pltpu.VMEM((2 * H, H), bf16),
            pltpu.VMEM((2 * H, 2 * H), bf16),
            pltpu.VMEM((1, 2 * H), f32),
            pltpu.VMEM((2 * H, H), bf16),
            pltpu.VMEM((H, H), bf16),
            pltpu.VMEM((H, H), bf16),
            pltpu.VMEM((N, H), bf16),
            pltpu.VMEM((H, 1), bf16),
        ],
        compiler_params=pltpu.CompilerParams(
            dimension_semantics=("arbitrary",),
        ),
        interpret=interpret,
    )(*ops)


def kernel(p, r, edge_index, pq_rel, params):
    B, L = p.shape
    E = edge_index.shape[1]
    Q = pq_rel.shape[1]
    H = params["q_emb"].shape[1]
    f32 = jnp.float32

    n = B * L
    n_pad = ((n + 255) // 256) * 256
    idx = jnp.concatenate(
        [p.reshape(-1).astype(jnp.int32),
         jnp.zeros((n_pad - n,), jnp.int32)])
    rows = _sc_gather_rows(pq_rel.astype(f32), idx)
    q1ht = jnp.transpose(rows[:n].reshape(B, L, Q), (1, 2, 0))

    rt = r.T.astype(f32).reshape(L, 1, B)
    ei = edge_index.astype(jnp.int32)
    ei_t = ei.T
    pr = params
    row = lambda a: a.reshape(1, -1)
    ops = (q1ht, rt, ei_t, ei,
           pr["q_emb"], pr["xq_emb"], pr["init_h"], pr["bias"].reshape(Q, 1),
           pr["mlp_self"]["W1"], row(pr["mlp_self"]["b1"]),
           pr["mlp_self"]["W2"], row(pr["mlp_self"]["b2"]),
           pr["mlp_outgo"]["W1"], row(pr["mlp_outgo"]["b1"]),
           pr["mlp_outgo"]["W2"], row(pr["mlp_outgo"]["b2"]),
           pr["mlp_income"]["W1"], row(pr["mlp_income"]["b1"]),
           pr["mlp_income"]["W2"], row(pr["mlp_income"]["b2"]),
           pr["gru"]["Wi"], row(pr["gru"]["bi"]),
           pr["gru"]["Wh"], row(pr["gru"]["bh"]),
           pr["mlp_pred"]["W1"], row(pr["mlp_pred"]["b1"]),
           pr["mlp_pred"]["W2"], row(pr["mlp_pred"]["b2"]), pr["w_out"])
    out = _run_scan(L, Q, B, E, H, ops)
    return jnp.transpose(out.reshape(L, Q, B), (2, 0, 1))
